# Initial kernel scaffold; baseline (speedup 1.0000x reference)
import jax
import jax.numpy as jnp
from jax.experimental import pallas as pl


def kernel(z_e, weight):
    raise NotImplementedError("write your pallas kernel here")


# speedup vs baseline: 1.2114x; 1.2114x over previous
import functools

import jax
import jax.numpy as jnp
from jax import lax
from jax.experimental import pallas as pl
from jax.experimental.pallas import tpu as pltpu, tpu_sc as plsc

N = 8192
K = 8192
D = 32
NW = 32
BPW = N // NW


DP = 128


def _sc_gather(table_hbm, idx_hbm, zq_hbm, idx_v, rows_v, sem):
    wid = lax.axis_index("s") * 2 + lax.axis_index("c")
    base = wid * BPW
    pltpu.sync_copy(idx_hbm.at[pl.ds(base, BPW)], idx_v)
    pltpu.async_copy(table_hbm.at[idx_v], rows_v, sem).wait()
    pltpu.sync_copy(rows_v, zq_hbm.at[pl.ds(base, BPW)])


@jax.jit
def _sc_call(weight_pad, idx):
    mesh = plsc.VectorSubcoreMesh(core_axis_name="c", subcore_axis_name="s")
    return pl.kernel(
        _sc_gather,
        mesh=mesh,
        out_type=jax.ShapeDtypeStruct((N, DP), jnp.float32),
        scratch_types=[
            pltpu.VMEM((BPW,), jnp.int32),
            pltpu.VMEM((BPW, DP), jnp.float32),
            pltpu.SemaphoreType.DMA,
        ],
    )(weight_pad, idx)


def _hist_perplexity_body(idx_ref, out_ref):
    ids = idx_ref[...]
    hi = ids >> 7
    lo = ids & 127
    qi = lax.broadcasted_iota(jnp.int32, (1, K // 128), 1)
    ri = lax.broadcasted_iota(jnp.int32, (1, 128), 1)
    e_hi = (hi == qi).astype(jnp.float32)
    e_lo = (lo == ri).astype(jnp.float32)
    counts = jax.lax.dot_general(
        e_hi, e_lo, dimension_numbers=(((0,), (0,)), ((), ())),
        preferred_element_type=jnp.float32)
    avg = counts * (1.0 / N)
    ent = avg * jnp.log(avg + 1e-10)
    out_ref[...] = jnp.exp(-jnp.sum(ent)).reshape(1, 1)


def _hist_perplexity(idx2d):
    return pl.pallas_call(
        _hist_perplexity_body,
        out_shape=jax.ShapeDtypeStruct((1, 1), jnp.float32),
    )(idx2d)


def kernel(z_e, weight):
    input_shape = z_e.shape
    flat_input = z_e.reshape(-1, D)
    distances = (jnp.sum(flat_input ** 2, axis=1, keepdims=True)
                 + jnp.sum(weight ** 2, axis=1)
                 - 2.0 * jnp.matmul(flat_input, weight.T))
    encoding_indices = jnp.argmin(distances, axis=1).astype(jnp.int32)
    weight_pad = jnp.pad(weight, ((0, 0), (0, DP - D)))
    z_q_flat = _sc_call(weight_pad, encoding_indices)[:, :D]
    perplexity = _hist_perplexity(encoding_indices.reshape(N, 1)).reshape(())
    return (z_q_flat.reshape(input_shape), perplexity)
